# Initial kernel scaffold; baseline (speedup 1.0000x reference)
import jax
import jax.numpy as jnp
from jax.experimental import pallas as pl


def kernel(agent_pos, agent_vel, pheremone_lattice, W1, b1, W2, b2):
    raise NotImplementedError("write your pallas kernel here")


# speedup vs baseline: 43.2885x; 43.2885x over previous
import functools
import math

import jax
import jax.numpy as jnp
from jax import lax
from jax.experimental import pallas as pl
from jax.experimental.pallas import tpu as pltpu
from jax.experimental.pallas import tpu_sc as plsc

N = 500000
G = 1024
C = 4
W8 = 8
GG = G * G
DT = 0.1
SA = 0.6
SL = 3.0
DECAY = 0.99

NW = 32
NPAD = 507904
CHUNK = NPAD // NW
SUB = 512
NSUB = CHUNK // SUB
TRASH = GG

_COS_SA = math.cos(SA)
_SIN_SA = math.sin(SA)

_BA = 128
_BI = NPAD // 128
_BM = 5000

_SCP = pltpu.CompilerParams(use_tc_tiling_on_sc=False)


def _box_decay_body(cur, top, bot, s_out, dec_out):
    x = cur[...]
    xs = x + jnp.concatenate([top[7:8], x[:-1]], axis=0)
    xs = xs + jnp.concatenate([x[1:], bot[0:1]], axis=0)
    ys = xs + pltpu.roll(xs, W8, 1) + pltpu.roll(xs, W8 * G - W8, 1)
    s_out[...] = ys + pltpu.roll(x, C, 1)
    dec_out[...] = x * DECAY


def _box_decay(lat8v):
    grid = G // _BA
    return pl.pallas_call(
        _box_decay_body,
        grid=(grid,),
        in_specs=[
            pl.BlockSpec((_BA, W8 * G), lambda i: (i, 0)),
            pl.BlockSpec((8, W8 * G), lambda i: ((i * (_BA // 8) - 1) % (G // 8), 0)),
            pl.BlockSpec((8, W8 * G), lambda i: (((i + 1) * (_BA // 8)) % (G // 8), 0)),
        ],
        out_specs=[
            pl.BlockSpec((_BA, W8 * G), lambda i: (i, 0)),
            pl.BlockSpec((_BA, W8 * G), lambda i: (i, 0)),
        ],
        out_shape=[
            jax.ShapeDtypeStruct((G, W8 * G), jnp.float32),
            jax.ShapeDtypeStruct((G + 1, W8 * G), jnp.float32),
        ],
    )(lat8v, lat8v, lat8v)


def _indices_body(pos, vel, sens0, sens1, sens2, depg, deps):
    i = pl.program_id(0)
    px = pos[0:1, :]
    py = pos[1:2, :]
    vx = vel[0:1, :]
    vy = vel[1:2, :]
    r = jnp.sqrt(vx * vx + vy * vy)
    safe = r > 0.0
    inv = jnp.where(safe, 1.0 / r, 0.0)
    cth = jnp.where(safe, vx * inv, 1.0)
    sth = jnp.where(safe, vy * inv, 0.0)
    gid = i * _BI + lax.broadcasted_iota(jnp.int32, (1, _BI), 1)
    valid = gid < N
    rows = []
    for co, so in ((1.0, 0.0), (_COS_SA, _SIN_SA), (_COS_SA, -_SIN_SA)):
        ca = cth * co - sth * so
        sa = sth * co + cth * so
        fx = jnp.rint(px + SL * ca).astype(jnp.int32)
        fy = jnp.rint(py + SL * sa).astype(jnp.int32)
        fx = fx + jnp.where(fx < 0, G, 0) - jnp.where(fx >= G, G, 0)
        fy = fy + jnp.where(fy < 0, G, 0) - jnp.where(fy >= G, G, 0)
        rows.append(jnp.where(valid, fx * G + fy, 0))
    sens0[...] = rows[0]
    sens1[...] = rows[1]
    sens2[...] = rows[2]
    dx = jnp.rint(px).astype(jnp.int32)
    dy = jnp.rint(py).astype(jnp.int32)
    cell = dx * G + dy
    depg[...] = jnp.where(valid, cell, 0)
    deps[...] = jnp.where(valid, cell, TRASH)


def _indices(posp, velp):
    grid = NPAD // _BI
    return pl.pallas_call(
        _indices_body,
        grid=(grid,),
        in_specs=[
            pl.BlockSpec((2, _BI), lambda i: (0, i)),
            pl.BlockSpec((2, _BI), lambda i: (0, i)),
        ],
        out_specs=[pl.BlockSpec((1, _BI), lambda i: (0, i))] * 5,
        out_shape=[jax.ShapeDtypeStruct((1, NPAD), jnp.int32)] * 5,
    )(posp, velp)


def _gather_body(tab, sens0, sens1, sens2, depg,
                 w0, w1, w2, old, idx_v, rows_v, sem):
    wid = lax.axis_index("s") * 2 + lax.axis_index("c")
    base = wid * CHUNK

    def one_table(idx_slice, out_hbm, start):
        pltpu.sync_copy(idx_slice, idx_v)
        handles = []
        for t in range(SUB // 128):
            handles.append(
                pltpu.async_copy(
                    tab.at[idx_v.at[pl.ds(t * 128, 128)]],
                    rows_v.at[pl.ds(t * 128, 128)],
                    sem,
                )
            )
        for h in handles:
            h.wait()
        pltpu.sync_copy(rows_v, out_hbm.at[pl.ds(start, SUB)])

    @pl.loop(0, NSUB)
    def _sub(j):
        start = base + j * SUB
        one_table(sens0.at[pl.ds(start, SUB)], w0, start)
        one_table(sens1.at[pl.ds(start, SUB)], w1, start)
        one_table(sens2.at[pl.ds(start, SUB)], w2, start)
        one_table(depg.at[pl.ds(start, SUB)], old, start)


def _gather(tab8, sens0, sens1, sens2, depg):
    mesh = plsc.VectorSubcoreMesh(core_axis_name="c", subcore_axis_name="s")
    row = jax.ShapeDtypeStruct((NPAD, W8), jnp.float32)
    return pl.kernel(
        _gather_body,
        out_type=(row, row, row, row),
        mesh=mesh,
        scratch_types=(
            pltpu.VMEM((SUB,), jnp.int32),
            pltpu.VMEM((SUB, W8), jnp.float32),
            pltpu.SemaphoreType.DMA,
        ),
        compiler_params=_SCP,
    )(tab8, sens0, sens1, sens2, depg)


def _mlp_body(w0, w1, w2, old, posT, w1m, b1m, w2m, b2m, nposT, velT, depv):
    inp = jnp.concatenate(
        [w0[:, 0:C], w1[:, 0:C], w2[:, 0:C]], axis=1
    )
    h = jnp.tanh(
        jnp.dot(inp, w1m[...], preferred_element_type=jnp.float32) + b1m[...]
    )
    o = jnp.dot(h, w2m[...], preferred_element_type=jnp.float32) + b2m[...]
    v = jnp.tanh(o[:, 0:2])
    d = o[:, 2:6]
    velT[...] = v
    p = posT[...] + v * DT
    nposT[...] = p - jnp.floor(p * (1.0 / G)) * G
    vals = DECAY * jnp.maximum(old[:, C:W8] + DT * d, 0.0)
    depv[...] = jnp.concatenate(
        [vals, jnp.zeros((_BM, C), jnp.float32)], axis=1
    )


def _mlp(w0, w1, w2, old, posT, W1, b1, W2, b2):
    grid = N // _BM
    row8 = pl.BlockSpec((_BM, W8), lambda i: (i, 0))
    row2 = pl.BlockSpec((_BM, 2), lambda i: (i, 0))
    full = lambda a, b: pl.BlockSpec((a, b), lambda i: (0, 0))
    return pl.pallas_call(
        _mlp_body,
        grid=(grid,),
        in_specs=[
            row8, row8, row8, row8, row2,
            full(12, 64), full(1, 64), full(64, 6), full(1, 6),
        ],
        out_specs=[row2, row2, row8],
        out_shape=[
            jax.ShapeDtypeStruct((N, 2), jnp.float32),
            jax.ShapeDtypeStruct((N, 2), jnp.float32),
            jax.ShapeDtypeStruct((NPAD, W8), jnp.float32),
        ],
    )(w0, w1, w2, old, posT, W1, b1.reshape(1, 64), W2, b2.reshape(1, 6))


_RB = 4


def _scatter_body(dep2d, depv, dec, idx_v, vals_v, sem):
    wid = lax.axis_index("s") * 2 + lax.axis_index("c")
    r0 = wid * (CHUNK // 128)

    @pl.loop(0, NSUB)
    def _sub(j):
        r = r0 + j * _RB
        pltpu.sync_copy(dep2d.at[pl.ds(r, _RB)], idx_v)
        pltpu.sync_copy(depv.at[pl.ds(r * 128, _RB * 128)], vals_v)
        handles = []
        for t in range(_RB):
            handles.append(
                pltpu.async_copy(
                    vals_v.at[pl.ds(t * 128, 128)],
                    dec.at[idx_v.at[t]],
                    sem,
                )
            )
        for h in handles:
            h.wait()


def _scatter(dep2d, depv, dec_ref):
    mesh = plsc.VectorSubcoreMesh(core_axis_name="c", subcore_axis_name="s")
    pl.kernel(
        _scatter_body,
        out_type=(),
        mesh=mesh,
        scratch_types=(
            pltpu.VMEM((_RB, 128), jnp.int32),
            pltpu.VMEM((_RB * 128, W8), jnp.float32),
            pltpu.SemaphoreType.DMA,
        ),
        compiler_params=_SCP,
    )(dep2d, depv, dec_ref)


def kernel(agent_pos, agent_vel, pheremone_lattice, W1, b1, W2, b2):
    lat8_t = jnp.concatenate(
        [
            jnp.transpose(pheremone_lattice.reshape(C, GG)),
            jnp.zeros((GG, C), jnp.float32),
        ],
        axis=1,
    )
    lat8v = lat8_t.reshape(G, W8 * G)

    tab8v, dec8v = _box_decay(lat8v)
    tab8 = tab8v.reshape(GG, W8)

    posp = jnp.pad(agent_pos, ((0, 0), (0, NPAD - N)))
    velp = jnp.pad(agent_vel, ((0, 0), (0, NPAD - N)))
    sens0, sens1, sens2, depg, deps = _indices(posp, velp)

    w0, w1, w2, old = _gather(
        tab8,
        sens0.reshape(NPAD), sens1.reshape(NPAD),
        sens2.reshape(NPAD), depg.reshape(NPAD),
    )

    posT = jnp.transpose(agent_pos)
    nposT, velT, depv = _mlp(w0, w1, w2, old, posT, W1, b1, W2, b2)

    dec_ref = jax.new_ref(dec8v.reshape(GG + G, W8))
    _scatter(deps.reshape(NPAD // 128, 128), depv, dec_ref)
    final_t = dec_ref[...]

    new_pos = jnp.transpose(nposT)
    new_vel = jnp.transpose(velT)
    new_lat = jnp.transpose(final_t[:GG, 0:C]).reshape(C, G, G)
    return (new_pos, new_vel, new_lat)


# speedup vs baseline: 67.3401x; 1.5556x over previous
import functools
import math

import jax
import jax.numpy as jnp
from jax import lax
from jax.experimental import pallas as pl
from jax.experimental.pallas import tpu as pltpu
from jax.experimental.pallas import tpu_sc as plsc

N = 500000
G = 1024
C = 4
W8 = 8
GG = G * G
DT = 0.1
SA = 0.6
SL = 3.0
DECAY = 0.99

NW = 32
NPAD = 507904
CHUNK = NPAD // NW
SUB = 512
NSUB = CHUNK // SUB
TRASH = GG

_COS_SA = math.cos(SA)
_SIN_SA = math.sin(SA)

_BA = 128
_BI = NPAD // 128
_BM = 5000

_SCP = pltpu.CompilerParams(use_tc_tiling_on_sc=False)


_BR = 8


def _box_decay_body(cur, top, bot, tab_out, dec_out):
    x = cur[...]
    xs = x + jnp.concatenate([top[:, 7:8], x[:, :-1]], axis=1)
    xs = xs + jnp.concatenate([x[:, 1:], bot[:, 0:1]], axis=1)
    ys = xs + pltpu.roll(xs, 1, 2) + pltpu.roll(xs, G - 1, 2)
    s_t = jnp.transpose(ys.reshape(C, _BR * G))
    o_t = jnp.transpose(x.reshape(C, _BR * G))
    tab_out[...] = jnp.concatenate([s_t, o_t], axis=1)
    dec_out[...] = jnp.concatenate(
        [DECAY * o_t, jnp.zeros((_BR * G, C), jnp.float32)], axis=1
    )


def _box_decay(lattice):
    grid = G // _BR
    halo = lambda off: pl.BlockSpec(
        (C, _BR, G), lambda i, o=off: (0, (i + o) % (G // _BR), 0)
    )
    return pl.pallas_call(
        _box_decay_body,
        grid=(grid,),
        in_specs=[halo(0), halo(-1), halo(1)],
        out_specs=[
            pl.BlockSpec((_BR * G, W8), lambda i: (i, 0)),
            pl.BlockSpec((_BR * G, W8), lambda i: (i, 0)),
        ],
        out_shape=[
            jax.ShapeDtypeStruct((GG, W8), jnp.float32),
            jax.ShapeDtypeStruct((GG + G, W8), jnp.float32),
        ],
    )(lattice, lattice, lattice)


def _untranspose_body(fin, out):
    x = fin[...][:, 0:C]
    out[...] = jnp.transpose(x).reshape(C, _BR, G)


def _untranspose(final8):
    return pl.pallas_call(
        _untranspose_body,
        grid=(G // _BR,),
        in_specs=[pl.BlockSpec((_BR * G, W8), lambda i: (i, 0))],
        out_specs=pl.BlockSpec((C, _BR, G), lambda i: (0, i, 0)),
        out_shape=jax.ShapeDtypeStruct((C, G, G), jnp.float32),
    )(final8)


def _indices_body(pos, vel, sens0, sens1, sens2, depg, deps):
    i = pl.program_id(0)
    px = pos[0:1, :]
    py = pos[1:2, :]
    vx = vel[0:1, :]
    vy = vel[1:2, :]
    r = jnp.sqrt(vx * vx + vy * vy)
    safe = r > 0.0
    inv = jnp.where(safe, 1.0 / r, 0.0)
    cth = jnp.where(safe, vx * inv, 1.0)
    sth = jnp.where(safe, vy * inv, 0.0)
    gid = i * _BI + lax.broadcasted_iota(jnp.int32, (1, _BI), 1)
    valid = gid < N
    rows = []
    for co, so in ((1.0, 0.0), (_COS_SA, _SIN_SA), (_COS_SA, -_SIN_SA)):
        ca = cth * co - sth * so
        sa = sth * co + cth * so
        fx = jnp.rint(px + SL * ca).astype(jnp.int32)
        fy = jnp.rint(py + SL * sa).astype(jnp.int32)
        fx = fx + jnp.where(fx < 0, G, 0) - jnp.where(fx >= G, G, 0)
        fy = fy + jnp.where(fy < 0, G, 0) - jnp.where(fy >= G, G, 0)
        rows.append(jnp.where(valid, fx * G + fy, 0))
    sens0[...] = rows[0]
    sens1[...] = rows[1]
    sens2[...] = rows[2]
    dx = jnp.rint(px).astype(jnp.int32)
    dy = jnp.rint(py).astype(jnp.int32)
    cell = dx * G + dy
    depg[...] = jnp.where(valid, cell, 0)
    deps[...] = jnp.where(valid, cell, TRASH)


def _indices(posp, velp):
    grid = NPAD // _BI
    return pl.pallas_call(
        _indices_body,
        grid=(grid,),
        in_specs=[
            pl.BlockSpec((2, _BI), lambda i: (0, i)),
            pl.BlockSpec((2, _BI), lambda i: (0, i)),
        ],
        out_specs=[pl.BlockSpec((1, _BI), lambda i: (0, i))] * 5,
        out_shape=[jax.ShapeDtypeStruct((1, NPAD), jnp.int32)] * 5,
    )(posp, velp)


def _gather_body(tab, sens0, sens1, sens2, depg,
                 w0, w1, w2, old, idx_v, rows_v, sem):
    wid = lax.axis_index("s") * 2 + lax.axis_index("c")
    base = wid * CHUNK

    def one_table(idx_slice, out_hbm, start):
        pltpu.sync_copy(idx_slice, idx_v)
        handles = []
        for t in range(SUB // 128):
            handles.append(
                pltpu.async_copy(
                    tab.at[idx_v.at[pl.ds(t * 128, 128)]],
                    rows_v.at[pl.ds(t * 128, 128)],
                    sem,
                )
            )
        for h in handles:
            h.wait()
        pltpu.sync_copy(rows_v, out_hbm.at[pl.ds(start, SUB)])

    @pl.loop(0, NSUB)
    def _sub(j):
        start = base + j * SUB
        one_table(sens0.at[pl.ds(start, SUB)], w0, start)
        one_table(sens1.at[pl.ds(start, SUB)], w1, start)
        one_table(sens2.at[pl.ds(start, SUB)], w2, start)
        one_table(depg.at[pl.ds(start, SUB)], old, start)


def _gather(tab8, sens0, sens1, sens2, depg):
    mesh = plsc.VectorSubcoreMesh(core_axis_name="c", subcore_axis_name="s")
    row = jax.ShapeDtypeStruct((NPAD, W8), jnp.float32)
    return pl.kernel(
        _gather_body,
        out_type=(row, row, row, row),
        mesh=mesh,
        scratch_types=(
            pltpu.VMEM((SUB,), jnp.int32),
            pltpu.VMEM((SUB, W8), jnp.float32),
            pltpu.SemaphoreType.DMA,
        ),
        compiler_params=_SCP,
    )(tab8, sens0, sens1, sens2, depg)


def _mlp_body(w0, w1, w2, old, posT, w1m, b1m, w2m, b2m, nposT, velT, depv):
    inp = jnp.concatenate(
        [w0[:, 0:C], w1[:, 0:C], w2[:, 0:C]], axis=1
    )
    h = jnp.tanh(
        jnp.dot(inp, w1m[...], preferred_element_type=jnp.float32) + b1m[...]
    )
    o = jnp.dot(h, w2m[...], preferred_element_type=jnp.float32) + b2m[...]
    v = jnp.tanh(o[:, 0:2])
    d = o[:, 2:6]
    velT[...] = v
    p = posT[...] + v * DT
    nposT[...] = p - jnp.floor(p * (1.0 / G)) * G
    vals = DECAY * jnp.maximum(old[:, C:W8] + DT * d, 0.0)
    depv[...] = jnp.concatenate(
        [vals, jnp.zeros((_BM, C), jnp.float32)], axis=1
    )


def _mlp(w0, w1, w2, old, posT, W1, b1, W2, b2):
    grid = N // _BM
    row8 = pl.BlockSpec((_BM, W8), lambda i: (i, 0))
    row2 = pl.BlockSpec((_BM, 2), lambda i: (i, 0))
    full = lambda a, b: pl.BlockSpec((a, b), lambda i: (0, 0))
    return pl.pallas_call(
        _mlp_body,
        grid=(grid,),
        in_specs=[
            row8, row8, row8, row8, row2,
            full(12, 64), full(1, 64), full(64, 6), full(1, 6),
        ],
        out_specs=[row2, row2, row8],
        out_shape=[
            jax.ShapeDtypeStruct((N, 2), jnp.float32),
            jax.ShapeDtypeStruct((N, 2), jnp.float32),
            jax.ShapeDtypeStruct((NPAD, W8), jnp.float32),
        ],
    )(w0, w1, w2, old, posT, W1, b1.reshape(1, 64), W2, b2.reshape(1, 6))


_RB = 4


def _scatter_body(dep2d, depv, dec, idx_v, vals_v, sem):
    wid = lax.axis_index("s") * 2 + lax.axis_index("c")
    r0 = wid * (CHUNK // 128)

    @pl.loop(0, NSUB)
    def _sub(j):
        r = r0 + j * _RB
        pltpu.sync_copy(dep2d.at[pl.ds(r, _RB)], idx_v)
        pltpu.sync_copy(depv.at[pl.ds(r * 128, _RB * 128)], vals_v)
        handles = []
        for t in range(_RB):
            handles.append(
                pltpu.async_copy(
                    vals_v.at[pl.ds(t * 128, 128)],
                    dec.at[idx_v.at[t]],
                    sem,
                )
            )
        for h in handles:
            h.wait()


def _scatter(dep2d, depv, dec_ref):
    mesh = plsc.VectorSubcoreMesh(core_axis_name="c", subcore_axis_name="s")
    pl.kernel(
        _scatter_body,
        out_type=(),
        mesh=mesh,
        scratch_types=(
            pltpu.VMEM((_RB, 128), jnp.int32),
            pltpu.VMEM((_RB * 128, W8), jnp.float32),
            pltpu.SemaphoreType.DMA,
        ),
        compiler_params=_SCP,
    )(dep2d, depv, dec_ref)


def kernel(agent_pos, agent_vel, pheremone_lattice, W1, b1, W2, b2):
    tab8, dec8 = _box_decay(pheremone_lattice)

    posp = jnp.pad(agent_pos, ((0, 0), (0, NPAD - N)))
    velp = jnp.pad(agent_vel, ((0, 0), (0, NPAD - N)))
    sens0, sens1, sens2, depg, deps = _indices(posp, velp)

    w0, w1, w2, old = _gather(
        tab8,
        sens0.reshape(NPAD), sens1.reshape(NPAD),
        sens2.reshape(NPAD), depg.reshape(NPAD),
    )

    posT = jnp.transpose(agent_pos)
    nposT, velT, depv = _mlp(w0, w1, w2, old, posT, W1, b1, W2, b2)

    dec_ref = jax.new_ref(dec8)
    _scatter(deps.reshape(NPAD // 128, 128), depv, dec_ref)
    final_t = dec_ref[...]

    new_pos = jnp.transpose(nposT)
    new_vel = jnp.transpose(velT)
    new_lat = _untranspose(final_t)
    return (new_pos, new_vel, new_lat)


# speedup vs baseline: 76.7981x; 1.1405x over previous
import functools
import math

import jax
import jax.numpy as jnp
from jax import lax
from jax.experimental import pallas as pl
from jax.experimental.pallas import tpu as pltpu
from jax.experimental.pallas import tpu_sc as plsc

N = 500000
G = 1024
C = 4
W8 = 8
GG = G * G
DT = 0.1
SA = 0.6
SL = 3.0
DECAY = 0.99

NW = 32
NPAD = 507904
CHUNK = NPAD // NW
SUB = 512
NSUB = CHUNK // SUB
TRASH = GG

_COS_SA = math.cos(SA)
_SIN_SA = math.sin(SA)

_BA = 128
_BI = NPAD // 128
_BM = 4096

_SCP = pltpu.CompilerParams(use_tc_tiling_on_sc=False)


_BR = 16


def _box_decay_body(cur, top, bot, tab_out, dec_out):
    x = cur[...]
    xs = x + jnp.concatenate([top[:, 7:8], x[:, :-1]], axis=1)
    xs = xs + jnp.concatenate([x[:, 1:], bot[:, 0:1]], axis=1)
    ys = xs + pltpu.roll(xs, 1, 2) + pltpu.roll(xs, G - 1, 2)
    s_t = jnp.transpose(ys.reshape(C, _BR * G))
    o_t = jnp.transpose(x.reshape(C, _BR * G))
    tab_out[...] = jnp.concatenate([s_t, o_t], axis=1)
    dec_out[...] = jnp.concatenate(
        [DECAY * o_t, jnp.zeros((_BR * G, C), jnp.float32)], axis=1
    )


def _box_decay(lattice):
    grid = G // _BR
    nb8 = G // 8
    r8 = _BR // 8
    return pl.pallas_call(
        _box_decay_body,
        grid=(grid,),
        in_specs=[
            pl.BlockSpec((C, _BR, G), lambda i: (0, i, 0)),
            pl.BlockSpec((C, 8, G), lambda i: (0, (i * r8 - 1) % nb8, 0)),
            pl.BlockSpec((C, 8, G), lambda i: (0, ((i + 1) * r8) % nb8, 0)),
        ],
        out_specs=[
            pl.BlockSpec((_BR * G, W8), lambda i: (i, 0)),
            pl.BlockSpec((_BR * G, W8), lambda i: (i, 0)),
        ],
        out_shape=[
            jax.ShapeDtypeStruct((GG, W8), jnp.float32),
            jax.ShapeDtypeStruct((GG + G, W8), jnp.float32),
        ],
    )(lattice, lattice, lattice)


def _untranspose_body(fin, out):
    x = fin[...][:, 0:C]
    out[...] = jnp.transpose(x).reshape(C, _BR, G)


def _untranspose(final8):
    return pl.pallas_call(
        _untranspose_body,
        grid=(G // _BR,),
        in_specs=[pl.BlockSpec((_BR * G, W8), lambda i: (i, 0))],
        out_specs=pl.BlockSpec((C, _BR, G), lambda i: (0, i, 0)),
        out_shape=jax.ShapeDtypeStruct((C, G, G), jnp.float32),
    )(final8)


def _indices_body(pos, vel, sens0, sens1, sens2, depg, deps):
    i = pl.program_id(0)
    px = pos[0:1, :]
    py = pos[1:2, :]
    vx = vel[0:1, :]
    vy = vel[1:2, :]
    r = jnp.sqrt(vx * vx + vy * vy)
    safe = r > 0.0
    inv = jnp.where(safe, 1.0 / r, 0.0)
    cth = jnp.where(safe, vx * inv, 1.0)
    sth = jnp.where(safe, vy * inv, 0.0)
    gid = i * _BI + lax.broadcasted_iota(jnp.int32, (1, _BI), 1)
    valid = gid < N
    rows = []
    for co, so in ((1.0, 0.0), (_COS_SA, _SIN_SA), (_COS_SA, -_SIN_SA)):
        ca = cth * co - sth * so
        sa = sth * co + cth * so
        fx = jnp.rint(px + SL * ca).astype(jnp.int32)
        fy = jnp.rint(py + SL * sa).astype(jnp.int32)
        fx = fx + jnp.where(fx < 0, G, 0) - jnp.where(fx >= G, G, 0)
        fy = fy + jnp.where(fy < 0, G, 0) - jnp.where(fy >= G, G, 0)
        rows.append(jnp.where(valid, fx * G + fy, 0))
    sens0[...] = rows[0]
    sens1[...] = rows[1]
    sens2[...] = rows[2]
    dx = jnp.rint(px).astype(jnp.int32)
    dy = jnp.rint(py).astype(jnp.int32)
    cell = dx * G + dy
    depg[...] = jnp.where(valid, cell, 0)
    deps[...] = jnp.where(valid, cell, TRASH)


def _indices(posp, velp):
    grid = NPAD // _BI
    return pl.pallas_call(
        _indices_body,
        grid=(grid,),
        in_specs=[
            pl.BlockSpec((2, _BI), lambda i: (0, i)),
            pl.BlockSpec((2, _BI), lambda i: (0, i)),
        ],
        out_specs=[pl.BlockSpec((1, _BI), lambda i: (0, i))] * 5,
        out_shape=[jax.ShapeDtypeStruct((1, NPAD), jnp.int32)] * 5,
    )(posp, velp)


def _gather_body(tab, sens0, sens1, sens2, depg,
                 w0, w1, w2, old, idx_v, rows_v, sem):
    wid = lax.axis_index("s") * 2 + lax.axis_index("c")
    base = wid * CHUNK

    def one_table(idx_slice, out_hbm, start):
        pltpu.sync_copy(idx_slice, idx_v)
        handles = []
        for t in range(SUB // 128):
            handles.append(
                pltpu.async_copy(
                    tab.at[idx_v.at[pl.ds(t * 128, 128)]],
                    rows_v.at[pl.ds(t * 128, 128)],
                    sem,
                )
            )
        for h in handles:
            h.wait()
        pltpu.sync_copy(rows_v, out_hbm.at[pl.ds(start, SUB)])

    @pl.loop(0, NSUB)
    def _sub(j):
        start = base + j * SUB
        one_table(sens0.at[pl.ds(start, SUB)], w0, start)
        one_table(sens1.at[pl.ds(start, SUB)], w1, start)
        one_table(sens2.at[pl.ds(start, SUB)], w2, start)
        one_table(depg.at[pl.ds(start, SUB)], old, start)


def _gather(tab8, sens0, sens1, sens2, depg):
    mesh = plsc.VectorSubcoreMesh(core_axis_name="c", subcore_axis_name="s")
    row = jax.ShapeDtypeStruct((NPAD, W8), jnp.float32)
    return pl.kernel(
        _gather_body,
        out_type=(row, row, row, row),
        mesh=mesh,
        scratch_types=(
            pltpu.VMEM((SUB,), jnp.int32),
            pltpu.VMEM((SUB, W8), jnp.float32),
            pltpu.SemaphoreType.DMA,
        ),
        compiler_params=_SCP,
    )(tab8, sens0, sens1, sens2, depg)


def _mlp_body(w0, w1, w2, old, pos, w1m, b1m, w2m, b2m, npos, vel, depv):
    inp = jnp.concatenate(
        [w0[:, 0:C], w1[:, 0:C], w2[:, 0:C]], axis=1
    )
    h = jnp.tanh(
        jnp.dot(inp, w1m[...], preferred_element_type=jnp.float32) + b1m[...]
    )
    o = jnp.dot(h, w2m[...], preferred_element_type=jnp.float32) + b2m[...]
    v = jnp.transpose(jnp.tanh(o[:, 0:2]))
    d = o[:, 2:6]
    vel[...] = v
    p = pos[...] + v * DT
    npos[...] = p - jnp.floor(p * (1.0 / G)) * G
    vals = DECAY * jnp.maximum(old[:, C:W8] + DT * d, 0.0)
    depv[...] = jnp.concatenate(
        [vals, jnp.zeros((_BM, C), jnp.float32)], axis=1
    )


def _mlp(w0, w1, w2, old, pos, W1, b1, W2, b2):
    grid = (N + _BM - 1) // _BM
    row8 = pl.BlockSpec((_BM, W8), lambda i: (i, 0))
    col2 = pl.BlockSpec((2, _BM), lambda i: (0, i))
    full = lambda a, b: pl.BlockSpec((a, b), lambda i: (0, 0))
    return pl.pallas_call(
        _mlp_body,
        grid=(grid,),
        in_specs=[
            row8, row8, row8, row8, col2,
            full(12, 64), full(1, 64), full(64, 6), full(1, 6),
        ],
        out_specs=[col2, col2, row8],
        out_shape=[
            jax.ShapeDtypeStruct((2, N), jnp.float32),
            jax.ShapeDtypeStruct((2, N), jnp.float32),
            jax.ShapeDtypeStruct((NPAD, W8), jnp.float32),
        ],
    )(w0, w1, w2, old, pos, W1, b1.reshape(1, 64), W2, b2.reshape(1, 6))


_RB = 4


def _scatter_body(dep2d, depv, dec, idx_v, vals_v, sem):
    wid = lax.axis_index("s") * 2 + lax.axis_index("c")
    r0 = wid * (CHUNK // 128)

    @pl.loop(0, NSUB)
    def _sub(j):
        r = r0 + j * _RB
        pltpu.sync_copy(dep2d.at[pl.ds(r, _RB)], idx_v)
        pltpu.sync_copy(depv.at[pl.ds(r * 128, _RB * 128)], vals_v)
        handles = []
        for t in range(_RB):
            handles.append(
                pltpu.async_copy(
                    vals_v.at[pl.ds(t * 128, 128)],
                    dec.at[idx_v.at[t]],
                    sem,
                )
            )
        for h in handles:
            h.wait()


def _scatter(dep2d, depv, dec_ref):
    mesh = plsc.VectorSubcoreMesh(core_axis_name="c", subcore_axis_name="s")
    pl.kernel(
        _scatter_body,
        out_type=(),
        mesh=mesh,
        scratch_types=(
            pltpu.VMEM((_RB, 128), jnp.int32),
            pltpu.VMEM((_RB * 128, W8), jnp.float32),
            pltpu.SemaphoreType.DMA,
        ),
        compiler_params=_SCP,
    )(dep2d, depv, dec_ref)


def kernel(agent_pos, agent_vel, pheremone_lattice, W1, b1, W2, b2):
    tab8, dec8 = _box_decay(pheremone_lattice)

    posp = jnp.pad(agent_pos, ((0, 0), (0, NPAD - N)))
    velp = jnp.pad(agent_vel, ((0, 0), (0, NPAD - N)))
    sens0, sens1, sens2, depg, deps = _indices(posp, velp)

    w0, w1, w2, old = _gather(
        tab8,
        sens0.reshape(NPAD), sens1.reshape(NPAD),
        sens2.reshape(NPAD), depg.reshape(NPAD),
    )

    new_pos, new_vel, depv = _mlp(w0, w1, w2, old, agent_pos, W1, b1, W2, b2)

    dec_ref = jax.new_ref(dec8)
    _scatter(deps.reshape(NPAD // 128, 128), depv, dec_ref)
    final_t = dec_ref[...]

    new_lat = _untranspose(final_t)
    return (new_pos, new_vel, new_lat)


# speedup vs baseline: 80.9936x; 1.0546x over previous
import functools
import math

import jax
import jax.numpy as jnp
from jax import lax
from jax.experimental import pallas as pl
from jax.experimental.pallas import tpu as pltpu
from jax.experimental.pallas import tpu_sc as plsc

N = 500000
G = 1024
C = 4
W8 = 8
GG = G * G
DT = 0.1
SA = 0.6
SL = 3.0
DECAY = 0.99

NW = 32
NPAD = 507904
CHUNK = NPAD // NW
SUB = 512
NSUB = CHUNK // SUB
TRASH = GG

_COS_SA = math.cos(SA)
_SIN_SA = math.sin(SA)

_BA = 128
_BI = NPAD // 128
_BM = 4096

_SCP = pltpu.CompilerParams(use_tc_tiling_on_sc=False)


_BR = 16


_MF = _BR * G // 16


def _pack_flat(z):
    t = jnp.transpose(z.reshape(W8, _MF, 16), (1, 2, 0))
    return t.reshape(_MF, 128)


def _box_decay_body(cur, top, bot, tab_out, dec_out):
    x = cur[...]
    xs = x + jnp.concatenate([top[:, 7:8], x[:, :-1]], axis=1)
    xs = xs + jnp.concatenate([x[:, 1:], bot[:, 0:1]], axis=1)
    ys = xs + pltpu.roll(xs, 1, 2) + pltpu.roll(xs, G - 1, 2)
    ys2 = ys.reshape(C, _BR * G)
    x2 = x.reshape(C, _BR * G)
    zt = jnp.concatenate([ys2, x2], axis=0)
    zd = jnp.concatenate(
        [DECAY * x2, jnp.zeros((C, _BR * G), jnp.float32)], axis=0
    )
    tab_out[...] = _pack_flat(zt)
    dec_out[...] = _pack_flat(zd)


def _box_decay(lattice):
    grid = G // _BR
    nb8 = G // 8
    r8 = _BR // 8
    return pl.pallas_call(
        _box_decay_body,
        grid=(grid,),
        in_specs=[
            pl.BlockSpec((C, _BR, G), lambda i: (0, i, 0)),
            pl.BlockSpec((C, 8, G), lambda i: (0, (i * r8 - 1) % nb8, 0)),
            pl.BlockSpec((C, 8, G), lambda i: (0, ((i + 1) * r8) % nb8, 0)),
        ],
        out_specs=[
            pl.BlockSpec((_MF, 128), lambda i: (i, 0)),
            pl.BlockSpec((_MF, 128), lambda i: (i, 0)),
        ],
        out_shape=[
            jax.ShapeDtypeStruct((GG * W8 // 128, 128), jnp.float32),
            jax.ShapeDtypeStruct(((GG + G) * W8 // 128, 128), jnp.float32),
        ],
    )(lattice, lattice, lattice)


def _untranspose_body(fin, out):
    x = fin[...]
    t = jnp.transpose(x.reshape(_MF, 16, W8), (2, 0, 1))
    t2 = t.reshape(W8, _BR * G)[0:C]
    out[...] = t2.reshape(C, _BR, G)


def _untranspose(fin_flat):
    return pl.pallas_call(
        _untranspose_body,
        grid=(G // _BR,),
        in_specs=[pl.BlockSpec((_MF, 128), lambda i: (i, 0))],
        out_specs=pl.BlockSpec((C, _BR, G), lambda i: (0, i, 0)),
        out_shape=jax.ShapeDtypeStruct((C, G, G), jnp.float32),
    )(fin_flat)


def _indices_body(pos, vel, sens0, sens1, sens2, depg, deps):
    i = pl.program_id(0)
    px = pos[0:1, :]
    py = pos[1:2, :]
    vx = vel[0:1, :]
    vy = vel[1:2, :]
    r = jnp.sqrt(vx * vx + vy * vy)
    safe = r > 0.0
    inv = jnp.where(safe, 1.0 / r, 0.0)
    cth = jnp.where(safe, vx * inv, 1.0)
    sth = jnp.where(safe, vy * inv, 0.0)
    gid = i * _BI + lax.broadcasted_iota(jnp.int32, (1, _BI), 1)
    valid = gid < N
    rows = []
    for co, so in ((1.0, 0.0), (_COS_SA, _SIN_SA), (_COS_SA, -_SIN_SA)):
        ca = cth * co - sth * so
        sa = sth * co + cth * so
        fx = jnp.rint(px + SL * ca).astype(jnp.int32)
        fy = jnp.rint(py + SL * sa).astype(jnp.int32)
        fx = fx + jnp.where(fx < 0, G, 0) - jnp.where(fx >= G, G, 0)
        fy = fy + jnp.where(fy < 0, G, 0) - jnp.where(fy >= G, G, 0)
        rows.append(jnp.where(valid, fx * G + fy, 0))
    sens0[...] = rows[0]
    sens1[...] = rows[1]
    sens2[...] = rows[2]
    dx = jnp.rint(px).astype(jnp.int32)
    dy = jnp.rint(py).astype(jnp.int32)
    cell = dx * G + dy
    depg[...] = jnp.where(valid, cell, 0)
    deps[...] = jnp.where(valid, cell, TRASH)


def _indices(posp, velp):
    grid = NPAD // _BI
    return pl.pallas_call(
        _indices_body,
        grid=(grid,),
        in_specs=[
            pl.BlockSpec((2, _BI), lambda i: (0, i)),
            pl.BlockSpec((2, _BI), lambda i: (0, i)),
        ],
        out_specs=[pl.BlockSpec((1, _BI), lambda i: (0, i))] * 5,
        out_shape=[jax.ShapeDtypeStruct((1, NPAD), jnp.int32)] * 5,
    )(posp, velp)


def _gather_body(tab, sens0, sens1, sens2, depg,
                 w0, w1, w2, old, idx_v, rows_v, sem):
    wid = lax.axis_index("s") * 2 + lax.axis_index("c")
    base = wid * CHUNK

    def one_table(idx_slice, out_hbm, start):
        pltpu.sync_copy(idx_slice, idx_v)
        handles = []
        for t in range(SUB // 128):
            handles.append(
                pltpu.async_copy(
                    tab.at[idx_v.at[pl.ds(t * 128, 128)]],
                    rows_v.at[pl.ds(t * 128, 128)],
                    sem,
                )
            )
        for h in handles:
            h.wait()
        pltpu.sync_copy(rows_v, out_hbm.at[pl.ds(start, SUB)])

    @pl.loop(0, NSUB)
    def _sub(j):
        start = base + j * SUB
        one_table(sens0.at[pl.ds(start, SUB)], w0, start)
        one_table(sens1.at[pl.ds(start, SUB)], w1, start)
        one_table(sens2.at[pl.ds(start, SUB)], w2, start)
        one_table(depg.at[pl.ds(start, SUB)], old, start)


def _gather(tab8, sens0, sens1, sens2, depg):
    mesh = plsc.VectorSubcoreMesh(core_axis_name="c", subcore_axis_name="s")
    row = jax.ShapeDtypeStruct((NPAD, W8), jnp.float32)
    return pl.kernel(
        _gather_body,
        out_type=(row, row, row, row),
        mesh=mesh,
        scratch_types=(
            pltpu.VMEM((SUB,), jnp.int32),
            pltpu.VMEM((SUB, W8), jnp.float32),
            pltpu.SemaphoreType.DMA,
        ),
        compiler_params=_SCP,
    )(tab8, sens0, sens1, sens2, depg)


def _mlp_body(w0, w1, w2, old, pos, w1m, b1m, w2m, b2m, npos, vel, depv):
    inp = jnp.concatenate(
        [w0[:, 0:C], w1[:, 0:C], w2[:, 0:C]], axis=1
    )
    h = jnp.tanh(
        jnp.dot(inp, w1m[...], preferred_element_type=jnp.float32) + b1m[...]
    )
    o = jnp.dot(h, w2m[...], preferred_element_type=jnp.float32) + b2m[...]
    v = jnp.transpose(jnp.tanh(o[:, 0:2]))
    d = o[:, 2:6]
    vel[...] = v
    p = pos[...] + v * DT
    npos[...] = p - jnp.floor(p * (1.0 / G)) * G
    vals = DECAY * jnp.maximum(old[:, C:W8] + DT * d, 0.0)
    depv[...] = jnp.concatenate(
        [vals, jnp.zeros((_BM, C), jnp.float32)], axis=1
    )


def _mlp(w0, w1, w2, old, pos, W1, b1, W2, b2):
    grid = (N + _BM - 1) // _BM
    row8 = pl.BlockSpec((_BM, W8), lambda i: (i, 0))
    col2 = pl.BlockSpec((2, _BM), lambda i: (0, i))
    full = lambda a, b: pl.BlockSpec((a, b), lambda i: (0, 0))
    return pl.pallas_call(
        _mlp_body,
        grid=(grid,),
        in_specs=[
            row8, row8, row8, row8, col2,
            full(12, 64), full(1, 64), full(64, 6), full(1, 6),
        ],
        out_specs=[col2, col2, row8],
        out_shape=[
            jax.ShapeDtypeStruct((2, N), jnp.float32),
            jax.ShapeDtypeStruct((2, N), jnp.float32),
            jax.ShapeDtypeStruct((NPAD, W8), jnp.float32),
        ],
    )(w0, w1, w2, old, pos, W1, b1.reshape(1, 64), W2, b2.reshape(1, 6))


_RB = 4


def _scatter_body(dep2d, depv, dec, idx_v, vals_v, sem):
    wid = lax.axis_index("s") * 2 + lax.axis_index("c")
    r0 = wid * (CHUNK // 128)

    @pl.loop(0, NSUB)
    def _sub(j):
        r = r0 + j * _RB
        pltpu.sync_copy(dep2d.at[pl.ds(r, _RB)], idx_v)
        pltpu.sync_copy(depv.at[pl.ds(r * 128, _RB * 128)], vals_v)
        handles = []
        for t in range(_RB):
            handles.append(
                pltpu.async_copy(
                    vals_v.at[pl.ds(t * 128, 128)],
                    dec.at[idx_v.at[t]],
                    sem,
                )
            )
        for h in handles:
            h.wait()


def _scatter(dep2d, depv, dec_ref):
    mesh = plsc.VectorSubcoreMesh(core_axis_name="c", subcore_axis_name="s")
    pl.kernel(
        _scatter_body,
        out_type=(),
        mesh=mesh,
        scratch_types=(
            pltpu.VMEM((_RB, 128), jnp.int32),
            pltpu.VMEM((_RB * 128, W8), jnp.float32),
            pltpu.SemaphoreType.DMA,
        ),
        compiler_params=_SCP,
    )(dep2d, depv, dec_ref)


def kernel(agent_pos, agent_vel, pheremone_lattice, W1, b1, W2, b2):
    tab_flat, dec_flat = _box_decay(pheremone_lattice)
    tab8 = tab_flat.reshape(GG, W8)
    dec8 = dec_flat.reshape(GG + G, W8)

    posp = jnp.pad(agent_pos, ((0, 0), (0, NPAD - N)))
    velp = jnp.pad(agent_vel, ((0, 0), (0, NPAD - N)))
    sens0, sens1, sens2, depg, deps = _indices(posp, velp)

    w0, w1, w2, old = _gather(
        tab8,
        sens0.reshape(NPAD), sens1.reshape(NPAD),
        sens2.reshape(NPAD), depg.reshape(NPAD),
    )

    new_pos, new_vel, depv = _mlp(w0, w1, w2, old, agent_pos, W1, b1, W2, b2)

    dec_ref = jax.new_ref(dec8)
    _scatter(deps.reshape(NPAD // 128, 128), depv, dec_ref)
    final_t = dec_ref[...]

    new_lat = _untranspose(final_t.reshape((GG + G) * W8 // 128, 128))
    return (new_pos, new_vel, new_lat)


# speedup vs baseline: 94.7163x; 1.1694x over previous
import functools
import math

import jax
import jax.numpy as jnp
from jax import lax
from jax.experimental import pallas as pl
from jax.experimental.pallas import tpu as pltpu
from jax.experimental.pallas import tpu_sc as plsc

N = 500000
G = 1024
C = 4
W8 = 8
GG = G * G
DT = 0.1
SA = 0.6
SL = 3.0
DECAY = 0.99

NW = 32
NPAD = 507904
CHUNK = NPAD // NW
SUB = 512
NSUB = CHUNK // SUB
TRASH = GG

_COS_SA = math.cos(SA)
_SIN_SA = math.sin(SA)

_BA = 128
_BI = NPAD // 128
_BM = 4096

_SCP = pltpu.CompilerParams(use_tc_tiling_on_sc=False)


_BR = 16


_MF = _BR * G // 16


def _pack_flat(z):
    t = jnp.transpose(z.reshape(W8, _MF, 16), (1, 2, 0))
    return t.reshape(_MF, 128)


def _box_decay_body(cur, top, bot, tab_out, dec_out):
    x = cur[...]
    xs = x + jnp.concatenate([top[:, 7:8], x[:, :-1]], axis=1)
    xs = xs + jnp.concatenate([x[:, 1:], bot[:, 0:1]], axis=1)
    ys = xs + pltpu.roll(xs, 1, 2) + pltpu.roll(xs, G - 1, 2)
    ys2 = ys.reshape(C, _BR * G)
    x2 = x.reshape(C, _BR * G)
    zt = jnp.concatenate([ys2, x2], axis=0)
    tab = _pack_flat(zt)
    tab_out[...] = tab
    lane = lax.broadcasted_iota(jnp.int32, (_MF, 128), 1)
    dec_out[...] = jnp.where(
        lane % W8 < C, DECAY * pltpu.roll(tab, 124, 1), 0.0
    )


def _box_decay(lattice):
    grid = G // _BR
    nb8 = G // 8
    r8 = _BR // 8
    return pl.pallas_call(
        _box_decay_body,
        grid=(grid,),
        in_specs=[
            pl.BlockSpec((C, _BR, G), lambda i: (0, i, 0)),
            pl.BlockSpec((C, 8, G), lambda i: (0, (i * r8 - 1) % nb8, 0)),
            pl.BlockSpec((C, 8, G), lambda i: (0, ((i + 1) * r8) % nb8, 0)),
        ],
        out_specs=[
            pl.BlockSpec((_MF, 128), lambda i: (i, 0)),
            pl.BlockSpec((_MF, 128), lambda i: (i, 0)),
        ],
        out_shape=[
            jax.ShapeDtypeStruct((GG * W8 // 128, 128), jnp.float32),
            jax.ShapeDtypeStruct(((GG + G) * W8 // 128, 128), jnp.float32),
        ],
    )(lattice, lattice, lattice)


def _untranspose_body(fin, out):
    x = fin[...]
    t = jnp.transpose(x.reshape(_MF, 16, W8), (2, 0, 1))
    t2 = t.reshape(W8, _BR * G)[0:C]
    out[...] = t2.reshape(C, _BR, G)


def _untranspose(fin_flat):
    return pl.pallas_call(
        _untranspose_body,
        grid=(G // _BR,),
        in_specs=[pl.BlockSpec((_MF, 128), lambda i: (i, 0))],
        out_specs=pl.BlockSpec((C, _BR, G), lambda i: (0, i, 0)),
        out_shape=jax.ShapeDtypeStruct((C, G, G), jnp.float32),
    )(fin_flat)


def _indices_body(pos, vel, sens0, sens1, sens2, depg, deps):
    i = pl.program_id(0)
    px = pos[0:1, :]
    py = pos[1:2, :]
    vx = vel[0:1, :]
    vy = vel[1:2, :]
    r = jnp.sqrt(vx * vx + vy * vy)
    safe = r > 0.0
    inv = jnp.where(safe, 1.0 / r, 0.0)
    cth = jnp.where(safe, vx * inv, 1.0)
    sth = jnp.where(safe, vy * inv, 0.0)
    gid = i * _BI + lax.broadcasted_iota(jnp.int32, (1, _BI), 1)
    valid = gid < N
    rows = []
    for co, so in ((1.0, 0.0), (_COS_SA, _SIN_SA), (_COS_SA, -_SIN_SA)):
        ca = cth * co - sth * so
        sa = sth * co + cth * so
        fx = jnp.rint(px + SL * ca).astype(jnp.int32)
        fy = jnp.rint(py + SL * sa).astype(jnp.int32)
        fx = fx + jnp.where(fx < 0, G, 0) - jnp.where(fx >= G, G, 0)
        fy = fy + jnp.where(fy < 0, G, 0) - jnp.where(fy >= G, G, 0)
        rows.append(jnp.where(valid, fx * G + fy, 0))
    sens0[...] = rows[0]
    sens1[...] = rows[1]
    sens2[...] = rows[2]
    dx = jnp.rint(px).astype(jnp.int32)
    dy = jnp.rint(py).astype(jnp.int32)
    cell = dx * G + dy
    depg[...] = jnp.where(valid, cell, 0)
    deps[...] = jnp.where(valid, cell, TRASH)


def _indices(posp, velp):
    grid = NPAD // _BI
    return pl.pallas_call(
        _indices_body,
        grid=(grid,),
        in_specs=[
            pl.BlockSpec((2, _BI), lambda i: (0, i)),
            pl.BlockSpec((2, _BI), lambda i: (0, i)),
        ],
        out_specs=[pl.BlockSpec((1, _BI), lambda i: (0, i))] * 5,
        out_shape=[jax.ShapeDtypeStruct((1, NPAD), jnp.int32)] * 5,
    )(posp, velp)


def _gather_body(tab, sens0, sens1, sens2, depg,
                 w0, w1, w2, old, idx_v, rows_v, sem):
    wid = lax.axis_index("s") * 2 + lax.axis_index("c")
    base = wid * CHUNK

    def one_table(idx_slice, out_hbm, start):
        pltpu.sync_copy(idx_slice, idx_v)
        handles = []
        for t in range(SUB // 128):
            handles.append(
                pltpu.async_copy(
                    tab.at[idx_v.at[pl.ds(t * 128, 128)]],
                    rows_v.at[pl.ds(t * 128, 128)],
                    sem,
                )
            )
        for h in handles:
            h.wait()
        pltpu.sync_copy(rows_v, out_hbm.at[pl.ds(start, SUB)])

    @pl.loop(0, NSUB)
    def _sub(j):
        start = base + j * SUB
        one_table(sens0.at[pl.ds(start, SUB)], w0, start)
        one_table(sens1.at[pl.ds(start, SUB)], w1, start)
        one_table(sens2.at[pl.ds(start, SUB)], w2, start)
        one_table(depg.at[pl.ds(start, SUB)], old, start)


def _gather(tab8, sens0, sens1, sens2, depg):
    mesh = plsc.VectorSubcoreMesh(core_axis_name="c", subcore_axis_name="s")
    row = jax.ShapeDtypeStruct((NPAD, W8), jnp.float32)
    return pl.kernel(
        _gather_body,
        out_type=(row, row, row, row),
        mesh=mesh,
        scratch_types=(
            pltpu.VMEM((SUB,), jnp.int32),
            pltpu.VMEM((SUB, W8), jnp.float32),
            pltpu.SemaphoreType.DMA,
        ),
        compiler_params=_SCP,
    )(tab8, sens0, sens1, sens2, depg)


def _mlp_body(w0, w1, w2, old, pos, w1m, b1m, w2m, b2m, npos, vel, depv):
    inp = jnp.concatenate(
        [w0[:, 0:C], w1[:, 0:C], w2[:, 0:C]], axis=1
    )
    h = jnp.tanh(
        jnp.dot(inp, w1m[...], preferred_element_type=jnp.float32) + b1m[...]
    )
    o = jnp.dot(h, w2m[...], preferred_element_type=jnp.float32) + b2m[...]
    v = jnp.transpose(jnp.tanh(o[:, 0:2]))
    d = o[:, 2:6]
    vel[...] = v
    p = pos[...] + v * DT
    npos[...] = p - jnp.floor(p * (1.0 / G)) * G
    vals = DECAY * jnp.maximum(old[:, C:W8] + DT * d, 0.0)
    depv[...] = jnp.concatenate(
        [vals, jnp.zeros((_BM, C), jnp.float32)], axis=1
    )


def _mlp(w0, w1, w2, old, pos, W1, b1, W2, b2):
    grid = (N + _BM - 1) // _BM
    row8 = pl.BlockSpec((_BM, W8), lambda i: (i, 0))
    col2 = pl.BlockSpec((2, _BM), lambda i: (0, i))
    full = lambda a, b: pl.BlockSpec((a, b), lambda i: (0, 0))
    return pl.pallas_call(
        _mlp_body,
        grid=(grid,),
        in_specs=[
            row8, row8, row8, row8, col2,
            full(12, 64), full(1, 64), full(64, 6), full(1, 6),
        ],
        out_specs=[col2, col2, row8],
        out_shape=[
            jax.ShapeDtypeStruct((2, N), jnp.float32),
            jax.ShapeDtypeStruct((2, N), jnp.float32),
            jax.ShapeDtypeStruct((NPAD, W8), jnp.float32),
        ],
    )(w0, w1, w2, old, pos, W1, b1.reshape(1, 64), W2, b2.reshape(1, 6))


_RB = 4


def _scatter_body(dep2d, depv, dec, idx_v, vals_v, sem):
    wid = lax.axis_index("s") * 2 + lax.axis_index("c")
    r0 = wid * (CHUNK // 128)

    @pl.loop(0, NSUB)
    def _sub(j):
        r = r0 + j * _RB
        pltpu.sync_copy(dep2d.at[pl.ds(r, _RB)], idx_v)
        pltpu.sync_copy(depv.at[pl.ds(r * 128, _RB * 128)], vals_v)
        handles = []
        for t in range(_RB):
            handles.append(
                pltpu.async_copy(
                    vals_v.at[pl.ds(t * 128, 128)],
                    dec.at[idx_v.at[t]],
                    sem,
                )
            )
        for h in handles:
            h.wait()


def _scatter(dep2d, depv, dec_ref):
    mesh = plsc.VectorSubcoreMesh(core_axis_name="c", subcore_axis_name="s")
    pl.kernel(
        _scatter_body,
        out_type=(),
        mesh=mesh,
        scratch_types=(
            pltpu.VMEM((_RB, 128), jnp.int32),
            pltpu.VMEM((_RB * 128, W8), jnp.float32),
            pltpu.SemaphoreType.DMA,
        ),
        compiler_params=_SCP,
    )(dep2d, depv, dec_ref)


def kernel(agent_pos, agent_vel, pheremone_lattice, W1, b1, W2, b2):
    tab_flat, dec_flat = _box_decay(pheremone_lattice)
    tab8 = tab_flat.reshape(GG, W8)
    dec8 = dec_flat.reshape(GG + G, W8)

    posp = jnp.pad(agent_pos, ((0, 0), (0, NPAD - N)))
    velp = jnp.pad(agent_vel, ((0, 0), (0, NPAD - N)))
    sens0, sens1, sens2, depg, deps = _indices(posp, velp)

    w0, w1, w2, old = _gather(
        tab8,
        sens0.reshape(NPAD), sens1.reshape(NPAD),
        sens2.reshape(NPAD), depg.reshape(NPAD),
    )

    new_pos, new_vel, depv = _mlp(w0, w1, w2, old, agent_pos, W1, b1, W2, b2)

    dec_ref = jax.new_ref(dec8)
    _scatter(deps.reshape(NPAD // 128, 128), depv, dec_ref)
    final_t = dec_ref[...]

    new_lat = _untranspose(final_t.reshape((GG + G) * W8 // 128, 128))
    return (new_pos, new_vel, new_lat)


# speedup vs baseline: 97.8428x; 1.0330x over previous
import functools
import math

import jax
import jax.numpy as jnp
from jax import lax
from jax.experimental import pallas as pl
from jax.experimental.pallas import tpu as pltpu
from jax.experimental.pallas import tpu_sc as plsc

N = 500000
G = 1024
C = 4
W8 = 8
GG = G * G
DT = 0.1
SA = 0.6
SL = 3.0
DECAY = 0.99

NW = 32
NPAD = 507904
CHUNK = NPAD // NW
SUB = 512
NSUB = CHUNK // SUB
TRASH = GG

_COS_SA = math.cos(SA)
_SIN_SA = math.sin(SA)

_BA = 128
_BI = NPAD // 128
_BM = 4096

_SCP = pltpu.CompilerParams(use_tc_tiling_on_sc=False)


_BR = 16


_MF = _BR * G // 16


def _pack_flat(z):
    t = jnp.transpose(z.reshape(W8, _MF, 16), (1, 2, 0))
    return t.reshape(_MF, 128)


def _box_decay_body(cur, top, bot, tab_out, dec_out):
    x = cur[...]
    xs = x + jnp.concatenate([top[:, 7:8], x[:, :-1]], axis=1)
    xs = xs + jnp.concatenate([x[:, 1:], bot[:, 0:1]], axis=1)
    ys = xs + pltpu.roll(xs, 1, 2) + pltpu.roll(xs, G - 1, 2)
    ys2 = ys.reshape(C, _BR * G)
    x2 = x.reshape(C, _BR * G)
    zt = jnp.concatenate([ys2, x2], axis=0)
    tab = _pack_flat(zt)
    tab_out[...] = tab
    lane = lax.broadcasted_iota(jnp.int32, (_MF, 128), 1)
    dec_out[...] = jnp.where(
        lane % W8 < C, DECAY * pltpu.roll(tab, 124, 1), 0.0
    )


def _box_decay(lattice):
    grid = G // _BR
    nb8 = G // 8
    r8 = _BR // 8
    return pl.pallas_call(
        _box_decay_body,
        grid=(grid,),
        in_specs=[
            pl.BlockSpec((C, _BR, G), lambda i: (0, i, 0)),
            pl.BlockSpec((C, 8, G), lambda i: (0, (i * r8 - 1) % nb8, 0)),
            pl.BlockSpec((C, 8, G), lambda i: (0, ((i + 1) * r8) % nb8, 0)),
        ],
        out_specs=[
            pl.BlockSpec((_MF, 128), lambda i: (i, 0)),
            pl.BlockSpec((_MF, 128), lambda i: (i, 0)),
        ],
        out_shape=[
            jax.ShapeDtypeStruct((GG * W8 // 128, 128), jnp.float32),
            jax.ShapeDtypeStruct(((GG + G) * W8 // 128, 128), jnp.float32),
        ],
    )(lattice, lattice, lattice)


def _untranspose_body(fin, out):
    x = fin[...]
    t = jnp.transpose(x.reshape(_MF, 16, W8), (2, 0, 1))
    t2 = t.reshape(W8, _BR * G)[0:C]
    out[...] = t2.reshape(C, _BR, G)


def _untranspose(fin_flat):
    return pl.pallas_call(
        _untranspose_body,
        grid=(G // _BR,),
        in_specs=[pl.BlockSpec((_MF, 128), lambda i: (i, 0))],
        out_specs=pl.BlockSpec((C, _BR, G), lambda i: (0, i, 0)),
        out_shape=jax.ShapeDtypeStruct((C, G, G), jnp.float32),
    )(fin_flat)


def _indices_body(pos, vel, sens0, sens1, sens2, depg, deps):
    i = pl.program_id(0)
    px = pos[0:1, :]
    py = pos[1:2, :]
    vx = vel[0:1, :]
    vy = vel[1:2, :]
    r = jnp.sqrt(vx * vx + vy * vy)
    safe = r > 0.0
    inv = jnp.where(safe, 1.0 / r, 0.0)
    cth = jnp.where(safe, vx * inv, 1.0)
    sth = jnp.where(safe, vy * inv, 0.0)
    gid = i * _BI + lax.broadcasted_iota(jnp.int32, (1, _BI), 1)
    valid = gid < N
    rows = []
    for co, so in ((1.0, 0.0), (_COS_SA, _SIN_SA), (_COS_SA, -_SIN_SA)):
        ca = cth * co - sth * so
        sa = sth * co + cth * so
        fx = jnp.rint(px + SL * ca).astype(jnp.int32)
        fy = jnp.rint(py + SL * sa).astype(jnp.int32)
        fx = fx + jnp.where(fx < 0, G, 0) - jnp.where(fx >= G, G, 0)
        fy = fy + jnp.where(fy < 0, G, 0) - jnp.where(fy >= G, G, 0)
        rows.append(jnp.where(valid, fx * G + fy, 0))
    sens0[...] = rows[0]
    sens1[...] = rows[1]
    sens2[...] = rows[2]
    dx = jnp.rint(px).astype(jnp.int32)
    dy = jnp.rint(py).astype(jnp.int32)
    cell = dx * G + dy
    depg[...] = jnp.where(valid, cell, 0)
    deps[...] = jnp.where(valid, cell, TRASH)


def _indices(posp, velp):
    grid = NPAD // _BI
    return pl.pallas_call(
        _indices_body,
        grid=(grid,),
        in_specs=[
            pl.BlockSpec((2, _BI), lambda i: (0, i)),
            pl.BlockSpec((2, _BI), lambda i: (0, i)),
        ],
        out_specs=[pl.BlockSpec((1, _BI), lambda i: (0, i))] * 5,
        out_shape=[jax.ShapeDtypeStruct((1, NPAD), jnp.int32)] * 5,
    )(posp, velp)


def _gather_body(tab, sens0, sens1, sens2, depg, w0, w1, w2, old,
                 i0, i1, i2, i3, r0, r1, r2, r3, semi, semg, semo):
    wid = lax.axis_index("s") * 2 + lax.axis_index("c")
    base = wid * CHUNK
    idx_bufs = (i0, i1, i2, i3)
    row_bufs = (r0, r1, r2, r3)
    srcs = (sens0, sens1, sens2, depg)
    outs = (w0, w1, w2, old)

    @pl.loop(0, NSUB)
    def _sub(j):
        start = base + j * SUB
        hi = [
            pltpu.async_copy(srcs[k].at[pl.ds(start, SUB)], idx_bufs[k], semi)
            for k in range(4)
        ]
        for h in hi:
            h.wait()
        hg = []
        for k in range(4):
            for t in range(SUB // 128):
                hg.append(
                    pltpu.async_copy(
                        tab.at[idx_bufs[k].at[pl.ds(t * 128, 128)]],
                        row_bufs[k].at[pl.ds(t * 128, 128)],
                        semg,
                    )
                )
        for h in hg:
            h.wait()
        ho = [
            pltpu.async_copy(row_bufs[k], outs[k].at[pl.ds(start, SUB)], semo)
            for k in range(4)
        ]
        for h in ho:
            h.wait()


def _gather(tab8, sens0, sens1, sens2, depg):
    mesh = plsc.VectorSubcoreMesh(core_axis_name="c", subcore_axis_name="s")
    row = jax.ShapeDtypeStruct((NPAD, W8), jnp.float32)
    return pl.kernel(
        _gather_body,
        out_type=(row, row, row, row),
        mesh=mesh,
        scratch_types=(
            pltpu.VMEM((SUB,), jnp.int32),
            pltpu.VMEM((SUB,), jnp.int32),
            pltpu.VMEM((SUB,), jnp.int32),
            pltpu.VMEM((SUB,), jnp.int32),
            pltpu.VMEM((SUB, W8), jnp.float32),
            pltpu.VMEM((SUB, W8), jnp.float32),
            pltpu.VMEM((SUB, W8), jnp.float32),
            pltpu.VMEM((SUB, W8), jnp.float32),
            pltpu.SemaphoreType.DMA,
            pltpu.SemaphoreType.DMA,
            pltpu.SemaphoreType.DMA,
        ),
        compiler_params=_SCP,
    )(tab8, sens0, sens1, sens2, depg)


def _mlp_body(w0, w1, w2, old, pos, w1m, b1m, w2m, b2m, npos, vel, depv):
    inp = jnp.concatenate(
        [w0[:, 0:C], w1[:, 0:C], w2[:, 0:C]], axis=1
    )
    h = jnp.tanh(
        jnp.dot(inp, w1m[...], preferred_element_type=jnp.float32) + b1m[...]
    )
    o = jnp.dot(h, w2m[...], preferred_element_type=jnp.float32) + b2m[...]
    v = jnp.transpose(jnp.tanh(o[:, 0:2]))
    d = o[:, 2:6]
    vel[...] = v
    p = pos[...] + v * DT
    npos[...] = p - jnp.floor(p * (1.0 / G)) * G
    vals = DECAY * jnp.maximum(old[:, C:W8] + DT * d, 0.0)
    depv[...] = jnp.concatenate(
        [vals, jnp.zeros((_BM, C), jnp.float32)], axis=1
    )


def _mlp(w0, w1, w2, old, pos, W1, b1, W2, b2):
    grid = (N + _BM - 1) // _BM
    row8 = pl.BlockSpec((_BM, W8), lambda i: (i, 0))
    col2 = pl.BlockSpec((2, _BM), lambda i: (0, i))
    full = lambda a, b: pl.BlockSpec((a, b), lambda i: (0, 0))
    return pl.pallas_call(
        _mlp_body,
        grid=(grid,),
        in_specs=[
            row8, row8, row8, row8, col2,
            full(12, 64), full(1, 64), full(64, 6), full(1, 6),
        ],
        out_specs=[col2, col2, row8],
        out_shape=[
            jax.ShapeDtypeStruct((2, N), jnp.float32),
            jax.ShapeDtypeStruct((2, N), jnp.float32),
            jax.ShapeDtypeStruct((NPAD, W8), jnp.float32),
        ],
    )(w0, w1, w2, old, pos, W1, b1.reshape(1, 64), W2, b2.reshape(1, 6))


_RB = 4


def _scatter_body(dep2d, depv, dec, idx_v, vals_v, sem):
    wid = lax.axis_index("s") * 2 + lax.axis_index("c")
    r0 = wid * (CHUNK // 128)

    @pl.loop(0, NSUB)
    def _sub(j):
        r = r0 + j * _RB
        pltpu.sync_copy(dep2d.at[pl.ds(r, _RB)], idx_v)
        pltpu.sync_copy(depv.at[pl.ds(r * 128, _RB * 128)], vals_v)
        handles = []
        for t in range(_RB):
            handles.append(
                pltpu.async_copy(
                    vals_v.at[pl.ds(t * 128, 128)],
                    dec.at[idx_v.at[t]],
                    sem,
                )
            )
        for h in handles:
            h.wait()


def _scatter(dep2d, depv, dec_ref):
    mesh = plsc.VectorSubcoreMesh(core_axis_name="c", subcore_axis_name="s")
    pl.kernel(
        _scatter_body,
        out_type=(),
        mesh=mesh,
        scratch_types=(
            pltpu.VMEM((_RB, 128), jnp.int32),
            pltpu.VMEM((_RB * 128, W8), jnp.float32),
            pltpu.SemaphoreType.DMA,
        ),
        compiler_params=_SCP,
    )(dep2d, depv, dec_ref)


def kernel(agent_pos, agent_vel, pheremone_lattice, W1, b1, W2, b2):
    tab_flat, dec_flat = _box_decay(pheremone_lattice)
    tab8 = tab_flat.reshape(GG, W8)
    dec8 = dec_flat.reshape(GG + G, W8)

    posp = jnp.pad(agent_pos, ((0, 0), (0, NPAD - N)))
    velp = jnp.pad(agent_vel, ((0, 0), (0, NPAD - N)))
    sens0, sens1, sens2, depg, deps = _indices(posp, velp)

    w0, w1, w2, old = _gather(
        tab8,
        sens0.reshape(NPAD), sens1.reshape(NPAD),
        sens2.reshape(NPAD), depg.reshape(NPAD),
    )

    new_pos, new_vel, depv = _mlp(w0, w1, w2, old, agent_pos, W1, b1, W2, b2)

    dec_ref = jax.new_ref(dec8)
    _scatter(deps.reshape(NPAD // 128, 128), depv, dec_ref)
    final_t = dec_ref[...]

    new_lat = _untranspose(final_t.reshape((GG + G) * W8 // 128, 128))
    return (new_pos, new_vel, new_lat)


# speedup vs baseline: 99.1588x; 1.0134x over previous
import functools
import math

import jax
import jax.numpy as jnp
from jax import lax
from jax.experimental import pallas as pl
from jax.experimental.pallas import tpu as pltpu
from jax.experimental.pallas import tpu_sc as plsc

N = 500000
G = 1024
C = 4
W8 = 8
GG = G * G
DT = 0.1
SA = 0.6
SL = 3.0
DECAY = 0.99

NW = 32
NPAD = 507904
CHUNK = NPAD // NW
SUB = 512
NSUB = CHUNK // SUB
TRASH = GG

_COS_SA = math.cos(SA)
_SIN_SA = math.sin(SA)

_BA = 128
_BI = NPAD // 128
_BM = 8192

_SCP = pltpu.CompilerParams(use_tc_tiling_on_sc=False)


_BR = 16


_MF = _BR * G // 16


def _pack_flat(z):
    t = jnp.transpose(z.reshape(W8, _MF, 16), (1, 2, 0))
    return t.reshape(_MF, 128)


def _box_decay_body(cur, top, bot, tab_out, dec_out):
    x = cur[...]
    xs = x + jnp.concatenate([top[:, 7:8], x[:, :-1]], axis=1)
    xs = xs + jnp.concatenate([x[:, 1:], bot[:, 0:1]], axis=1)
    ys = xs + pltpu.roll(xs, 1, 2) + pltpu.roll(xs, G - 1, 2)
    ys2 = ys.reshape(C, _BR * G)
    x2 = x.reshape(C, _BR * G)
    zt = jnp.concatenate([ys2, x2], axis=0)
    tab = _pack_flat(zt)
    tab_out[...] = tab
    lane = lax.broadcasted_iota(jnp.int32, (_MF, 128), 1)
    dec_out[...] = jnp.where(
        lane % W8 < C, DECAY * pltpu.roll(tab, 124, 1), 0.0
    )


def _box_decay(lattice):
    grid = G // _BR
    nb8 = G // 8
    r8 = _BR // 8
    return pl.pallas_call(
        _box_decay_body,
        grid=(grid,),
        in_specs=[
            pl.BlockSpec((C, _BR, G), lambda i: (0, i, 0)),
            pl.BlockSpec((C, 8, G), lambda i: (0, (i * r8 - 1) % nb8, 0)),
            pl.BlockSpec((C, 8, G), lambda i: (0, ((i + 1) * r8) % nb8, 0)),
        ],
        out_specs=[
            pl.BlockSpec((_MF, 128), lambda i: (i, 0)),
            pl.BlockSpec((_MF, 128), lambda i: (i, 0)),
        ],
        out_shape=[
            jax.ShapeDtypeStruct((GG * W8 // 128, 128), jnp.float32),
            jax.ShapeDtypeStruct(((GG + G) * W8 // 128, 128), jnp.float32),
        ],
    )(lattice, lattice, lattice)


def _untranspose_body(fin, out):
    x = fin[...]
    t = jnp.transpose(x.reshape(_MF, 16, W8), (2, 0, 1))
    t2 = t.reshape(W8, _BR * G)[0:C]
    out[...] = t2.reshape(C, _BR, G)


def _untranspose(fin_flat):
    return pl.pallas_call(
        _untranspose_body,
        grid=(G // _BR,),
        in_specs=[pl.BlockSpec((_MF, 128), lambda i: (i, 0))],
        out_specs=pl.BlockSpec((C, _BR, G), lambda i: (0, i, 0)),
        out_shape=jax.ShapeDtypeStruct((C, G, G), jnp.float32),
    )(fin_flat)


def _indices_body(pos, vel, sens0, sens1, sens2, depg, deps):
    i = pl.program_id(0)
    px = pos[0:1, :]
    py = pos[1:2, :]
    vx = vel[0:1, :]
    vy = vel[1:2, :]
    r = jnp.sqrt(vx * vx + vy * vy)
    safe = r > 0.0
    inv = jnp.where(safe, 1.0 / r, 0.0)
    cth = jnp.where(safe, vx * inv, 1.0)
    sth = jnp.where(safe, vy * inv, 0.0)
    gid = i * _BI + lax.broadcasted_iota(jnp.int32, (1, _BI), 1)
    valid = gid < N
    rows = []
    for co, so in ((1.0, 0.0), (_COS_SA, _SIN_SA), (_COS_SA, -_SIN_SA)):
        ca = cth * co - sth * so
        sa = sth * co + cth * so
        fx = jnp.rint(px + SL * ca).astype(jnp.int32)
        fy = jnp.rint(py + SL * sa).astype(jnp.int32)
        fx = fx + jnp.where(fx < 0, G, 0) - jnp.where(fx >= G, G, 0)
        fy = fy + jnp.where(fy < 0, G, 0) - jnp.where(fy >= G, G, 0)
        rows.append(jnp.where(valid, fx * G + fy, 0))
    sens0[...] = rows[0]
    sens1[...] = rows[1]
    sens2[...] = rows[2]
    dx = jnp.rint(px).astype(jnp.int32)
    dy = jnp.rint(py).astype(jnp.int32)
    cell = dx * G + dy
    depg[...] = jnp.where(valid, cell, 0)
    deps[...] = jnp.where(valid, cell, TRASH)


def _indices(posp, velp):
    grid = NPAD // _BI
    return pl.pallas_call(
        _indices_body,
        grid=(grid,),
        in_specs=[
            pl.BlockSpec((2, _BI), lambda i: (0, i)),
            pl.BlockSpec((2, _BI), lambda i: (0, i)),
        ],
        out_specs=[pl.BlockSpec((1, _BI), lambda i: (0, i))] * 5,
        out_shape=[jax.ShapeDtypeStruct((1, NPAD), jnp.int32)] * 5,
    )(posp, velp)


def _gather_body(tab, sens0, sens1, sens2, depg, w0, w1, w2, old,
                 i0, i1, i2, i3, r0, r1, r2, r3, semi, semg, semo):
    wid = lax.axis_index("s") * 2 + lax.axis_index("c")
    base = wid * CHUNK
    idx_bufs = (i0, i1, i2, i3)
    row_bufs = (r0, r1, r2, r3)
    srcs = (sens0, sens1, sens2, depg)
    outs = (w0, w1, w2, old)

    @pl.loop(0, NSUB)
    def _sub(j):
        start = base + j * SUB
        hi = [
            pltpu.async_copy(srcs[k].at[pl.ds(start, SUB)], idx_bufs[k], semi)
            for k in range(4)
        ]
        for h in hi:
            h.wait()
        hg = []
        for k in range(4):
            for t in range(SUB // 128):
                hg.append(
                    pltpu.async_copy(
                        tab.at[idx_bufs[k].at[pl.ds(t * 128, 128)]],
                        row_bufs[k].at[pl.ds(t * 128, 128)],
                        semg,
                    )
                )
        for h in hg:
            h.wait()
        ho = [
            pltpu.async_copy(row_bufs[k], outs[k].at[pl.ds(start, SUB)], semo)
            for k in range(4)
        ]
        for h in ho:
            h.wait()


def _gather(tab8, sens0, sens1, sens2, depg):
    mesh = plsc.VectorSubcoreMesh(core_axis_name="c", subcore_axis_name="s")
    row = jax.ShapeDtypeStruct((NPAD, W8), jnp.float32)
    return pl.kernel(
        _gather_body,
        out_type=(row, row, row, row),
        mesh=mesh,
        scratch_types=(
            pltpu.VMEM((SUB,), jnp.int32),
            pltpu.VMEM((SUB,), jnp.int32),
            pltpu.VMEM((SUB,), jnp.int32),
            pltpu.VMEM((SUB,), jnp.int32),
            pltpu.VMEM((SUB, W8), jnp.float32),
            pltpu.VMEM((SUB, W8), jnp.float32),
            pltpu.VMEM((SUB, W8), jnp.float32),
            pltpu.VMEM((SUB, W8), jnp.float32),
            pltpu.SemaphoreType.DMA,
            pltpu.SemaphoreType.DMA,
            pltpu.SemaphoreType.DMA,
        ),
        compiler_params=_SCP,
    )(tab8, sens0, sens1, sens2, depg)


def _mlp_body(w0, w1, w2, old, pos, w1m, b1m, w2m, b2m, npos, vel, depv):
    inp = jnp.concatenate(
        [w0[:, 0:C], w1[:, 0:C], w2[:, 0:C]], axis=1
    )
    h = jnp.tanh(
        jnp.dot(inp, w1m[...], preferred_element_type=jnp.float32) + b1m[...]
    )
    o = jnp.dot(h, w2m[...], preferred_element_type=jnp.float32) + b2m[...]
    v = jnp.transpose(jnp.tanh(o[:, 0:2]))
    d = o[:, 2:6]
    vel[...] = v
    p = pos[...] + v * DT
    npos[...] = p - jnp.floor(p * (1.0 / G)) * G
    vals = DECAY * jnp.maximum(old[:, C:W8] + DT * d, 0.0)
    depv[...] = jnp.concatenate(
        [vals, jnp.zeros((_BM, C), jnp.float32)], axis=1
    )


def _mlp(w0, w1, w2, old, pos, W1, b1, W2, b2):
    grid = (N + _BM - 1) // _BM
    row8 = pl.BlockSpec((_BM, W8), lambda i: (i, 0))
    col2 = pl.BlockSpec((2, _BM), lambda i: (0, i))
    full = lambda a, b: pl.BlockSpec((a, b), lambda i: (0, 0))
    return pl.pallas_call(
        _mlp_body,
        grid=(grid,),
        in_specs=[
            row8, row8, row8, row8, col2,
            full(12, 64), full(1, 64), full(64, 6), full(1, 6),
        ],
        out_specs=[col2, col2, row8],
        out_shape=[
            jax.ShapeDtypeStruct((2, N), jnp.float32),
            jax.ShapeDtypeStruct((2, N), jnp.float32),
            jax.ShapeDtypeStruct((NPAD, W8), jnp.float32),
        ],
    )(w0, w1, w2, old, pos, W1, b1.reshape(1, 64), W2, b2.reshape(1, 6))


_RB = 4


def _scatter_body(dep2d, depv, dec, idx_v, vals_v, sem):
    wid = lax.axis_index("s") * 2 + lax.axis_index("c")
    r0 = wid * (CHUNK // 128)

    @pl.loop(0, NSUB)
    def _sub(j):
        r = r0 + j * _RB
        pltpu.sync_copy(dep2d.at[pl.ds(r, _RB)], idx_v)
        pltpu.sync_copy(depv.at[pl.ds(r * 128, _RB * 128)], vals_v)
        handles = []
        for t in range(_RB):
            handles.append(
                pltpu.async_copy(
                    vals_v.at[pl.ds(t * 128, 128)],
                    dec.at[idx_v.at[t]],
                    sem,
                )
            )
        for h in handles:
            h.wait()


def _scatter(dep2d, depv, dec_ref):
    mesh = plsc.VectorSubcoreMesh(core_axis_name="c", subcore_axis_name="s")
    pl.kernel(
        _scatter_body,
        out_type=(),
        mesh=mesh,
        scratch_types=(
            pltpu.VMEM((_RB, 128), jnp.int32),
            pltpu.VMEM((_RB * 128, W8), jnp.float32),
            pltpu.SemaphoreType.DMA,
        ),
        compiler_params=_SCP,
    )(dep2d, depv, dec_ref)


def kernel(agent_pos, agent_vel, pheremone_lattice, W1, b1, W2, b2):
    tab_flat, dec_flat = _box_decay(pheremone_lattice)
    tab8 = tab_flat.reshape(GG, W8)
    dec8 = dec_flat.reshape(GG + G, W8)

    posp = jnp.pad(agent_pos, ((0, 0), (0, NPAD - N)))
    velp = jnp.pad(agent_vel, ((0, 0), (0, NPAD - N)))
    sens0, sens1, sens2, depg, deps = _indices(posp, velp)

    w0, w1, w2, old = _gather(
        tab8,
        sens0.reshape(NPAD), sens1.reshape(NPAD),
        sens2.reshape(NPAD), depg.reshape(NPAD),
    )

    new_pos, new_vel, depv = _mlp(w0, w1, w2, old, agent_pos, W1, b1, W2, b2)

    dec_ref = jax.new_ref(dec8)
    _scatter(deps.reshape(NPAD // 128, 128), depv, dec_ref)
    final_t = dec_ref[...]

    new_lat = _untranspose(final_t.reshape((GG + G) * W8 // 128, 128))
    return (new_pos, new_vel, new_lat)
